# initial kernel scaffold (unmeasured)
import jax
import jax.numpy as jnp
from jax import lax
from jax.experimental import pallas as pl
from jax.experimental.pallas import tpu as pltpu


def kernel(
    x,
):
    def body(*refs):
        pass

    out_shape = jax.ShapeDtypeStruct(..., jnp.float32)
    return pl.pallas_call(body, out_shape=out_shape)(...)


# baseline (device time: 44935 ns/iter reference)
import jax
import jax.numpy as jnp
from jax import lax
from jax.experimental import pallas as pl
from jax.experimental.pallas import tpu as pltpu


def kernel(x):
    m, n = x.shape

    def body(x_ref, out_ref, recv_buf, send_sems, recv_sems):
        me = lax.axis_index("i")
        p_x = me ^ 1
        p_y = (me & 4) | (3 - (me & 3))
        p_z = me ^ 4
        partners = [p_x, p_y, p_z]

        barrier_sem = pltpu.get_barrier_semaphore()
        for nbr in partners:
            pl.semaphore_signal(
                barrier_sem, inc=1,
                device_id=(nbr,), device_id_type=pl.DeviceIdType.MESH,
            )
        pl.semaphore_wait(barrier_sem, 3)

        out_ref[...] = x_ref[...]
        for s, partner in enumerate(partners):
            rdma = pltpu.make_async_remote_copy(
                src_ref=out_ref,
                dst_ref=recv_buf.at[s],
                send_sem=send_sems.at[s],
                recv_sem=recv_sems.at[s],
                device_id=(partner,),
                device_id_type=pl.DeviceIdType.MESH,
            )
            rdma.start()
            rdma.wait()
            out_ref[...] += recv_buf[s]

    return pl.pallas_call(
        body,
        out_shape=jax.ShapeDtypeStruct((m, n), x.dtype),
        in_specs=[pl.BlockSpec(memory_space=pltpu.VMEM)],
        out_specs=pl.BlockSpec(memory_space=pltpu.VMEM),
        scratch_shapes=[
            pltpu.VMEM((3, m, n), x.dtype),
            pltpu.SemaphoreType.DMA((3,)),
            pltpu.SemaphoreType.DMA((3,)),
        ],
        compiler_params=pltpu.CompilerParams(collective_id=0),
    )(x)


# device time: 35213 ns/iter; 1.2761x vs baseline; 1.2761x over previous
import jax
import jax.numpy as jnp
from jax import lax
from jax.experimental import pallas as pl
from jax.experimental.pallas import tpu as pltpu


def kernel(x):
    m, n = x.shape
    h1, h2, h3 = m // 2, m // 4, m // 8

    def body(x_ref, out_ref, rs_buf1, rs_buf2, rs_buf3, send_sems, recv_sems):
        me = lax.axis_index("i")
        p_x = me ^ 1
        p_y = (me & 4) | (3 - (me & 3))
        p_z = me ^ 4

        xb = (me ^ (me >> 1)) & 1
        yb = (me >> 1) & 1
        zb = (me >> 2) & 1

        barrier_sem = pltpu.get_barrier_semaphore()
        for nbr in (p_x, p_y, p_z):
            pl.semaphore_signal(
                barrier_sem, inc=1,
                device_id=(nbr,), device_id_type=pl.DeviceIdType.MESH,
            )
        pl.semaphore_wait(barrier_sem, 3)

        out_ref[...] = x_ref[...]

        s1 = h1 * xb
        s2 = s1 + h2 * yb
        s3 = s2 + h3 * zb

        rs_stages = [
            (p_x, h1 * (1 - xb), s1, h1, rs_buf1, 0),
            (p_y, s1 + h2 * (1 - yb), s2, h2, rs_buf2, 1),
            (p_z, s2 + h3 * (1 - zb), s3, h3, rs_buf3, 2),
        ]
        for partner, send_start, keep_start, half, buf, i in rs_stages:
            rdma = pltpu.make_async_remote_copy(
                src_ref=out_ref.at[pl.ds(send_start, half), :],
                dst_ref=buf,
                send_sem=send_sems.at[i],
                recv_sem=recv_sems.at[i],
                device_id=(partner,),
                device_id_type=pl.DeviceIdType.MESH,
            )
            rdma.start()
            rdma.wait()
            out_ref[pl.ds(keep_start, half), :] += buf[...]

        ag_stages = [
            (p_z, s3, h3, 3),
            (p_y, s2, h2, 4),
            (p_x, s1, h1, 5),
        ]
        for partner, start, size, i in ag_stages:
            rdma = pltpu.make_async_remote_copy(
                src_ref=out_ref.at[pl.ds(start, size), :],
                dst_ref=out_ref.at[pl.ds(start, size), :],
                send_sem=send_sems.at[i],
                recv_sem=recv_sems.at[i],
                device_id=(partner,),
                device_id_type=pl.DeviceIdType.MESH,
            )
            rdma.start()
            rdma.wait()

    return pl.pallas_call(
        body,
        out_shape=jax.ShapeDtypeStruct((m, n), x.dtype),
        in_specs=[pl.BlockSpec(memory_space=pltpu.VMEM)],
        out_specs=pl.BlockSpec(memory_space=pltpu.VMEM),
        scratch_shapes=[
            pltpu.VMEM((h1, n), x.dtype),
            pltpu.VMEM((h2, n), x.dtype),
            pltpu.VMEM((h3, n), x.dtype),
            pltpu.SemaphoreType.DMA((6,)),
            pltpu.SemaphoreType.DMA((6,)),
        ],
        compiler_params=pltpu.CompilerParams(collective_id=0),
    )(x)


# device time: 23113 ns/iter; 1.9441x vs baseline; 1.5235x over previous
import jax
import jax.numpy as jnp
from jax import lax
from jax.experimental import pallas as pl
from jax.experimental.pallas import tpu as pltpu

N_DEV = 8


def kernel(x):
    m, n = x.shape
    c = m // N_DEV

    def body(x_ref, out_ref, rs_buf, rs_send_sems, rs_recv_sems,
             ag_send_sems, ag_recv_sems):
        me = lax.axis_index("i")
        my_start = c * me

        barrier_sem = pltpu.get_barrier_semaphore()
        for d in range(1, N_DEV):
            q = lax.rem(me + d, N_DEV)
            pl.semaphore_signal(
                barrier_sem, inc=1,
                device_id=(q,), device_id_type=pl.DeviceIdType.MESH,
            )
        pl.semaphore_wait(barrier_sem, N_DEV - 1)

        rs = []
        for d in range(1, N_DEV):
            q = lax.rem(me + d, N_DEV)
            s = N_DEV - d
            rdma = pltpu.make_async_remote_copy(
                src_ref=x_ref.at[pl.ds(c * q, c), :],
                dst_ref=rs_buf.at[s],
                send_sem=rs_send_sems.at[d - 1],
                recv_sem=rs_recv_sems.at[s],
                device_id=(q,),
                device_id_type=pl.DeviceIdType.MESH,
            )
            rdma.start()
            rs.append(rdma)

        out_ref[pl.ds(my_start, c), :] = x_ref[pl.ds(my_start, c), :]
        for d in range(1, N_DEV):
            s = N_DEV - d
            rs[d - 1].wait_recv()
            out_ref[pl.ds(my_start, c), :] += rs_buf[s]

        ag = []
        for d in range(1, N_DEV):
            q = lax.rem(me + d, N_DEV)
            s = N_DEV - d
            rdma = pltpu.make_async_remote_copy(
                src_ref=out_ref.at[pl.ds(my_start, c), :],
                dst_ref=out_ref.at[pl.ds(my_start, c), :],
                send_sem=ag_send_sems.at[d - 1],
                recv_sem=ag_recv_sems.at[s],
                device_id=(q,),
                device_id_type=pl.DeviceIdType.MESH,
            )
            rdma.start()
            ag.append(rdma)

        for d in range(1, N_DEV):
            ag[d - 1].wait_recv()

        for r in rs:
            r.wait_send()
        for a in ag:
            a.wait_send()

    return pl.pallas_call(
        body,
        out_shape=jax.ShapeDtypeStruct((m, n), x.dtype),
        in_specs=[pl.BlockSpec(memory_space=pltpu.VMEM)],
        out_specs=pl.BlockSpec(memory_space=pltpu.VMEM),
        scratch_shapes=[
            pltpu.VMEM((N_DEV, c, n), x.dtype),
            pltpu.SemaphoreType.DMA((N_DEV - 1,)),
            pltpu.SemaphoreType.DMA((N_DEV,)),
            pltpu.SemaphoreType.DMA((N_DEV - 1,)),
            pltpu.SemaphoreType.DMA((N_DEV,)),
        ],
        compiler_params=pltpu.CompilerParams(collective_id=0),
    )(x)


# device time: 20811 ns/iter; 2.1592x vs baseline; 1.1106x over previous
import jax
import jax.numpy as jnp
from jax import lax
from jax.experimental import pallas as pl
from jax.experimental.pallas import tpu as pltpu

N_DEV = 8
K = 2


def kernel(x):
    m, n = x.shape
    c = m // N_DEV
    w = c // K

    def body(x_ref, out_ref, rs_buf, rs_send_sems, rs_recv_sems,
             ag_send_sems, ag_recv_sems):
        me = lax.axis_index("i")
        my_start = c * me

        barrier_sem = pltpu.get_barrier_semaphore()
        for d in range(1, N_DEV):
            q = lax.rem(me + d, N_DEV)
            pl.semaphore_signal(
                barrier_sem, inc=1,
                device_id=(q,), device_id_type=pl.DeviceIdType.MESH,
            )
        pl.semaphore_wait(barrier_sem, N_DEV - 1)

        rs = []
        for k in range(K):
            for d in range(1, N_DEV):
                q = lax.rem(me + d, N_DEV)
                s = N_DEV - d
                rdma = pltpu.make_async_remote_copy(
                    src_ref=x_ref.at[pl.ds(c * q + w * k, w), :],
                    dst_ref=rs_buf.at[k, s],
                    send_sem=rs_send_sems.at[k, d - 1],
                    recv_sem=rs_recv_sems.at[k, s],
                    device_id=(q,),
                    device_id_type=pl.DeviceIdType.MESH,
                )
                rdma.start()
                rs.append(rdma)

        out_ref[pl.ds(my_start, c), :] = x_ref[pl.ds(my_start, c), :]

        ag = []
        for k in range(K):
            wave_start = my_start + w * k
            for d in range(1, N_DEV):
                s = N_DEV - d
                rs[k * (N_DEV - 1) + d - 1].wait_recv()
                out_ref[pl.ds(wave_start, w), :] += rs_buf[k, s]
            for d in range(1, N_DEV):
                q = lax.rem(me + d, N_DEV)
                s = N_DEV - d
                rdma = pltpu.make_async_remote_copy(
                    src_ref=out_ref.at[pl.ds(wave_start, w), :],
                    dst_ref=out_ref.at[pl.ds(wave_start, w), :],
                    send_sem=ag_send_sems.at[k, d - 1],
                    recv_sem=ag_recv_sems.at[k, s],
                    device_id=(q,),
                    device_id_type=pl.DeviceIdType.MESH,
                )
                rdma.start()
                ag.append(rdma)

        for a in ag:
            a.wait_recv()
        for r in rs:
            r.wait_send()
        for a in ag:
            a.wait_send()

    return pl.pallas_call(
        body,
        out_shape=jax.ShapeDtypeStruct((m, n), x.dtype),
        in_specs=[pl.BlockSpec(memory_space=pltpu.VMEM)],
        out_specs=pl.BlockSpec(memory_space=pltpu.VMEM),
        scratch_shapes=[
            pltpu.VMEM((K, N_DEV, w, n), x.dtype),
            pltpu.SemaphoreType.DMA((K, N_DEV - 1)),
            pltpu.SemaphoreType.DMA((K, N_DEV)),
            pltpu.SemaphoreType.DMA((K, N_DEV - 1)),
            pltpu.SemaphoreType.DMA((K, N_DEV)),
        ],
        compiler_params=pltpu.CompilerParams(collective_id=0),
    )(x)
